# baseline (speedup 1.0000x reference)
import functools

import jax
import jax.numpy as jnp
from jax import lax
from jax.experimental import pallas as pl
from jax.experimental.pallas import tpu as pltpu
from jax.experimental.pallas import tpu_sc as plsc

D_MODEL = 768
LANES = 16
NVEC = D_MODEL // LANES
N_WORKERS = 32
CHUNK = 32
NBUF = 3


def _rsqrt(x):
    xi = lax.bitcast_convert_type(x, jnp.int32)
    yi = jnp.full((LANES,), 0x5F3759DF, jnp.int32) - (xi >> 1)
    y = lax.bitcast_convert_type(yi, jnp.float32)
    for _ in range(3):
        y = y * (1.5 - 0.5 * x * y * y)
    return y


_GATHER_DNUMS = lax.GatherDimensionNumbers(
    offset_dims=(), collapsed_slice_dims=(0,), start_index_map=(0,))


def _permute(v, idx):
    return lax.gather(v, idx[:, None], _GATHER_DNUMS, slice_sizes=(1,),
                      mode=lax.GatherScatterMode.PROMISE_IN_BOUNDS)


def _lane_sum(v):
    for s in (8, 4, 2, 1):
        idx = lax.iota(jnp.int32, LANES) ^ s
        v = v + _permute(v, idx)
    return v


def _make_sc_kernel(batch, seq):
    n_tokens = batch * seq
    sblock = seq // N_WORKERS
    nhalf = sblock // CHUNK
    n_chunks = batch * nhalf
    mesh = plsc.VectorSubcoreMesh(core_axis_name="c", subcore_axis_name="s")

    @functools.partial(
        pl.kernel,
        mesh=mesh,
        out_type=jax.ShapeDtypeStruct((n_tokens, D_MODEL), jnp.float32),
        scratch_types=[
            pltpu.VMEM((nhalf, batch, CHUNK), jnp.int32),
            pltpu.VMEM((NBUF, CHUNK, D_MODEL), jnp.float32),
            pltpu.VMEM((CHUNK, D_MODEL), jnp.float32),
            pltpu.VMEM((D_MODEL,), jnp.float32),
            pltpu.VMEM((D_MODEL,), jnp.float32),
            pltpu.VMEM((CHUNK, LANES), jnp.float32),
            pltpu.VMEM((CHUNK, LANES), jnp.float32),
            pltpu.SemaphoreType.DMA,
            pltpu.SemaphoreType.DMA,
            pltpu.SemaphoreType.DMA,
            pltpu.SemaphoreType.DMA,
            pltpu.SemaphoreType.DMA,
            pltpu.SemaphoreType.DMA,
        ],
    )
    def k(ids_hbm, w_hbm, pos_hbm, gamma_hbm, beta_hbm, out_hbm,
          idx_v, rows_v, pos_v, g_v, b_v, mean_v, inv_v,
          sg0, sg1, sg2, so0, so1, so2):
        sg = (sg0, sg1, sg2)
        so = (so0, so1, so2)
        nc = 2
        wid = lax.axis_index("s") * nc + lax.axis_index("c")
        sbase = wid * sblock

        pltpu.sync_copy(ids_hbm.at[wid], idx_v)
        pltpu.sync_copy(gamma_hbm, g_v)
        pltpu.sync_copy(beta_hbm, b_v)

        def load_pos(h):
            pltpu.sync_copy(
                pos_hbm.at[pl.ds(sbase + h * CHUNK, CHUNK)], pos_v)

        def gather(c):
            h, b = divmod(c, batch)
            return pltpu.async_copy(
                w_hbm.at[idx_v.at[h, b]], rows_v.at[c % NBUF], sg[c % NBUF])

        def make_stats_body(buf):
            def stats_body(r, _):
                acc = jnp.zeros((LANES,), jnp.float32)
                acc2 = jnp.zeros((LANES,), jnp.float32)
                for j in range(NVEC):
                    sl = pl.ds(j * LANES, LANES)
                    e = rows_v[buf, r, sl] + pos_v[r, sl]
                    rows_v[buf, r, sl] = e
                    acc = acc + e
                    acc2 = acc2 + e * e
                mean = _lane_sum(acc) * (1.0 / D_MODEL)
                m2 = _lane_sum(acc2) * (1.0 / D_MODEL)
                var = jnp.maximum(m2 - mean * mean, 0.0)
                mean_v[r, :] = mean
                inv_v[r, :] = _rsqrt(var + 1e-12)
                return 0
            return stats_body

        jhalf = NVEC // 2

        def make_norm_body(buf, jb, gregs, bregs):
            def norm_body(r, _):
                mean = mean_v[r, :]
                inv = inv_v[r, :]
                for j in range(jhalf):
                    sl = pl.ds((jb * jhalf + j) * LANES, LANES)
                    e = rows_v[buf, r, sl]
                    rows_v[buf, r, sl] = (e - mean) * inv * gregs[j] + bregs[j]
                return 0
            return norm_body

        g_cp = {c: gather(c) for c in range(min(2, n_chunks))}
        o_cp = {}
        for c in range(n_chunks):
            buf = c % NBUF
            h, b = divmod(c, batch)
            if b == 0:
                load_pos(h)
            g_cp.pop(c).wait()
            lax.fori_loop(0, CHUNK, make_stats_body(buf), 0)
            for jb in range(2):
                gregs = [g_v[pl.ds((jb * jhalf + j) * LANES, LANES)]
                         for j in range(jhalf)]
                bregs = [b_v[pl.ds((jb * jhalf + j) * LANES, LANES)]
                         for j in range(jhalf)]
                lax.fori_loop(0, CHUNK, make_norm_body(buf, jb, gregs, bregs), 0)
            tok = b * seq + sbase + h * CHUNK
            o_cp[c] = pltpu.async_copy(
                rows_v.at[buf], out_hbm.at[pl.ds(tok, CHUNK)], so[buf])
            nxt = c + 2
            if nxt < n_chunks:
                if nxt >= NBUF:
                    o_cp.pop(nxt - NBUF).wait()
                g_cp[nxt] = gather(nxt)
        for c in sorted(o_cp):
            o_cp[c].wait()

    return k


@jax.jit
def kernel(input_ids, W, pos, gamma, beta):
    batch, seq = input_ids.shape
    sblock = seq // N_WORKERS
    nhalf = sblock // CHUNK
    ids = input_ids.reshape(batch, N_WORKERS, nhalf, CHUNK)
    ids = ids.transpose(1, 2, 0, 3).astype(jnp.int32)
    sc = _make_sc_kernel(batch, seq)
    out = sc(ids, W, pos[0, :seq], gamma, beta)
    return out.reshape(batch, seq, D_MODEL)
